# initial kernel scaffold (unmeasured)
import jax
import jax.numpy as jnp
from jax import lax
from jax.experimental import pallas as pl
from jax.experimental.pallas import tpu as pltpu

N_DEV = 4
SCALE = 0.08838834764831843
BLK = 64


def kernel(x, Wq, K_ext, V_ext, Wo):
    B, Sq, D = x.shape
    _, Skv_l, Hq, Dh = K_ext.shape
    QT = 512
    n_qt = Sq // QT

    xb = x.reshape(Sq, D).astype(jnp.bfloat16)
    wq = Wq.astype(jnp.bfloat16)
    wo = Wo.astype(jnp.bfloat16)
    k = jnp.transpose(K_ext.reshape(Skv_l, Hq, Dh), (1, 0, 2)).astype(jnp.bfloat16)
    v = jnp.transpose(V_ext.reshape(Skv_l, Hq, Dh), (1, 0, 2)).astype(jnp.bfloat16)

    def body(x_ref, wq_ref, k_ref, v_ref, wo_ref, out_ref,
             o_comm, ml_comm, q_scr, ctx_scr, so, ro, sml, rml):
        my = lax.axis_index("i")
        left = lax.rem(my + (N_DEV - 1), N_DEV)
        right = lax.rem(my + 1, N_DEV)

        q = lax.dot_general(x_ref[...], wq_ref[...],
                            (((1,), (0,)), ((), ())),
                            preferred_element_type=jnp.float32)
        q_scr[...] = (q * SCALE).astype(jnp.bfloat16)

        kb_base = my * (Skv_l // BLK)

        for h in range(Hq):
            k_h = k_ref[h]
            v_h = v_ref[h]
            for t in range(n_qt):
                q_t = q_scr[pl.ds(t * QT, QT), pl.ds(h * Dh, Dh)]
                s = lax.dot_general(q_t, k_h, (((1,), (1,)), ((), ())),
                                    preferred_element_type=jnp.float32)
                qb = (lax.broadcasted_iota(jnp.int32, (QT, Skv_l), 0)
                      + t * QT) // BLK
                kb = lax.broadcasted_iota(jnp.int32, (QT, Skv_l), 1) // BLK \
                    + kb_base
                keep = (qb == kb) | (kb == 0) | (lax.rem(qb + kb, 3) == 0)
                s = jnp.where(keep, s, -1e9)
                m = jnp.max(s, axis=1, keepdims=True)
                p = jnp.exp(s - m)
                l = jnp.sum(p, axis=1, keepdims=True)
                o = lax.dot_general(p.astype(jnp.bfloat16), v_h,
                                    (((1,), (0,)), ((), ())),
                                    preferred_element_type=jnp.float32)
                o_comm[0, pl.ds(t * QT, QT), pl.ds(h * Dh, Dh)] = \
                    o.astype(jnp.bfloat16)
                ml_comm[0, pl.ds(t * QT, QT), pl.ds(h, 1)] = m
                ml_comm[0, pl.ds(t * QT, QT), pl.ds(Hq + h, 1)] = l

        barrier_sem = pltpu.get_barrier_semaphore()
        for nbr in (left, right):
            pl.semaphore_signal(barrier_sem, inc=1, device_id=(nbr,),
                                device_id_type=pl.DeviceIdType.MESH)
        pl.semaphore_wait(barrier_sem, 2)

        for hop in range(N_DEV - 1):
            rdma_o = pltpu.make_async_remote_copy(
                src_ref=o_comm.at[hop], dst_ref=o_comm.at[hop + 1],
                send_sem=so.at[hop], recv_sem=ro.at[hop + 1],
                device_id=(right,), device_id_type=pl.DeviceIdType.MESH)
            rdma_ml = pltpu.make_async_remote_copy(
                src_ref=ml_comm.at[hop], dst_ref=ml_comm.at[hop + 1],
                send_sem=sml.at[hop], recv_sem=rml.at[hop + 1],
                device_id=(right,), device_id_type=pl.DeviceIdType.MESH)
            rdma_o.start()
            rdma_ml.start()
            rdma_o.wait()
            rdma_ml.wait()

        for h in range(Hq):
            ms = [ml_comm[s_, :, pl.ds(h, 1)] for s_ in range(N_DEV)]
            M = jnp.maximum(jnp.maximum(ms[0], ms[1]),
                            jnp.maximum(ms[2], ms[3]))
            O = None
            L = None
            for s_ in range(N_DEV):
                w = jnp.exp(ms[s_] - M)
                lw = ml_comm[s_, :, pl.ds(Hq + h, 1)] * w
                ow = o_comm[s_, :, pl.ds(h * Dh, Dh)].astype(jnp.float32) * w
                L = lw if L is None else L + lw
                O = ow if O is None else O + ow
            ctx_scr[:, pl.ds(h * Dh, Dh)] = (O / L).astype(jnp.bfloat16)

        out_ref[...] = lax.dot_general(ctx_scr[...], wo_ref[...],
                                       (((1,), (0,)), ((), ())),
                                       preferred_element_type=jnp.float32)

    y = pl.pallas_call(
        body,
        out_shape=jax.ShapeDtypeStruct((Sq, D), jnp.float32),
        in_specs=[pl.BlockSpec(memory_space=pltpu.VMEM)] * 5,
        out_specs=pl.BlockSpec(memory_space=pltpu.VMEM),
        scratch_shapes=[
            pltpu.VMEM((N_DEV, Sq, D), jnp.bfloat16),
            pltpu.VMEM((N_DEV, Sq, 2 * Hq), jnp.float32),
            pltpu.VMEM((Sq, D), jnp.bfloat16),
            pltpu.VMEM((Sq, D), jnp.bfloat16),
            pltpu.SemaphoreType.DMA((N_DEV,)),
            pltpu.SemaphoreType.DMA((N_DEV,)),
            pltpu.SemaphoreType.DMA((N_DEV,)),
            pltpu.SemaphoreType.DMA((N_DEV,)),
        ],
        compiler_params=pltpu.CompilerParams(collective_id=0),
    )(xb, wq, k, v, wo)

    return y.reshape(B, Sq, D)


# baseline (device time: 347059 ns/iter reference)
import jax
import jax.numpy as jnp
from jax import lax
from jax.experimental import pallas as pl
from jax.experimental.pallas import tpu as pltpu

N_DEV = 4
SCALE = 0.08838834764831843
BLK = 64


def kernel(x, Wq, K_ext, V_ext, Wo):
    B, Sq, D = x.shape
    _, Skv_l, Hq, Dh = K_ext.shape
    QT = 256
    n_qt = Sq // QT

    q = ((x.reshape(Sq, D) @ Wq) * SCALE).astype(jnp.bfloat16)
    k = jnp.transpose(K_ext.reshape(Skv_l, Hq, Dh), (1, 0, 2)).astype(jnp.bfloat16)
    v = jnp.transpose(V_ext.reshape(Skv_l, Hq, Dh), (1, 0, 2)).astype(jnp.bfloat16)

    def body(q_ref, k_ref, v_ref, out_ref, o_comm, ml_comm, so, ro, sml, rml):
        my = lax.axis_index("i")
        left = lax.rem(my + (N_DEV - 1), N_DEV)
        right = lax.rem(my + 1, N_DEV)

        kb_base = my * (Skv_l // BLK)

        for h in range(Hq):
            def attn_step(t, carry, h=h):
                q_t = q_ref[pl.ds(t * QT, QT), h * Dh:(h + 1) * Dh]
                k_h = k_ref[h]
                v_h = v_ref[h]
                s = lax.dot_general(q_t, k_h, (((1,), (1,)), ((), ())),
                                    preferred_element_type=jnp.float32)
                qb = (lax.broadcasted_iota(jnp.int32, (QT, Skv_l), 0)
                      + t * QT) // BLK
                kb = lax.broadcasted_iota(jnp.int32, (QT, Skv_l), 1) // BLK \
                    + kb_base
                keep = (qb == kb) | (kb == 0) | (lax.rem(qb + kb, 3) == 0)
                s = jnp.where(keep, s, -1e9)
                m = jnp.max(s, axis=1, keepdims=True)
                p = jnp.exp(s - m)
                l = jnp.sum(p, axis=1, keepdims=True)
                o = lax.dot_general(p.astype(jnp.bfloat16), v_h,
                                    (((1,), (0,)), ((), ())),
                                    preferred_element_type=jnp.float32)
                o_comm[0, pl.ds(t * QT, QT), h * Dh:(h + 1) * Dh] = \
                    o.astype(jnp.bfloat16)
                ml_comm[0, pl.ds(t * QT, QT), h:h + 1] = m
                ml_comm[0, pl.ds(t * QT, QT), Hq + h:Hq + h + 1] = l
                return carry

            lax.fori_loop(0, n_qt, attn_step, 0)

        barrier_sem = pltpu.get_barrier_semaphore()
        for nbr in (left, right):
            pl.semaphore_signal(barrier_sem, inc=1, device_id=(nbr,),
                                device_id_type=pl.DeviceIdType.MESH)
        pl.semaphore_wait(barrier_sem, 2)

        for hop in range(N_DEV - 1):
            rdma_o = pltpu.make_async_remote_copy(
                src_ref=o_comm.at[hop], dst_ref=o_comm.at[hop + 1],
                send_sem=so.at[hop], recv_sem=ro.at[hop + 1],
                device_id=(right,), device_id_type=pl.DeviceIdType.MESH)
            rdma_ml = pltpu.make_async_remote_copy(
                src_ref=ml_comm.at[hop], dst_ref=ml_comm.at[hop + 1],
                send_sem=sml.at[hop], recv_sem=rml.at[hop + 1],
                device_id=(right,), device_id_type=pl.DeviceIdType.MESH)
            rdma_o.start()
            rdma_ml.start()
            rdma_o.wait()
            rdma_ml.wait()

        expand = (lax.broadcasted_iota(jnp.int32, (Hq, D), 1) // Dh ==
                  lax.broadcasted_iota(jnp.int32, (Hq, D), 0)
                  ).astype(jnp.float32)

        def comb_step(t, carry):
            rows = pl.ds(t * QT, QT)
            ms = [ml_comm[s_, rows, 0:Hq] for s_ in range(N_DEV)]
            M = jnp.maximum(jnp.maximum(ms[0], ms[1]),
                            jnp.maximum(ms[2], ms[3]))
            O = None
            L = None
            for s_ in range(N_DEV):
                w = jnp.exp(ms[s_] - M)
                lw = ml_comm[s_, rows, Hq:2 * Hq] * w
                wx = lax.dot_general(w, expand, (((1,), (0,)), ((), ())),
                                     preferred_element_type=jnp.float32)
                ow = o_comm[s_, rows, :].astype(jnp.float32) * wx
                L = lw if L is None else L + lw
                O = ow if O is None else O + ow
            Lx = lax.dot_general(L, expand, (((1,), (0,)), ((), ())),
                                 preferred_element_type=jnp.float32)
            out_ref[rows, :] = (O / Lx).astype(jnp.bfloat16)
            return carry

        lax.fori_loop(0, n_qt, comb_step, 0)

    ctx = pl.pallas_call(
        body,
        out_shape=jax.ShapeDtypeStruct((Sq, D), jnp.bfloat16),
        in_specs=[pl.BlockSpec(memory_space=pltpu.VMEM)] * 3,
        out_specs=pl.BlockSpec(memory_space=pltpu.VMEM),
        scratch_shapes=[
            pltpu.VMEM((N_DEV, Sq, D), jnp.bfloat16),
            pltpu.VMEM((N_DEV, Sq, 2 * Hq), jnp.float32),
            pltpu.SemaphoreType.DMA((N_DEV,)),
            pltpu.SemaphoreType.DMA((N_DEV,)),
            pltpu.SemaphoreType.DMA((N_DEV,)),
            pltpu.SemaphoreType.DMA((N_DEV,)),
        ],
        compiler_params=pltpu.CompilerParams(
            collective_id=0,
            vmem_limit_bytes=46 * 1024 * 1024,
        ),
    )(q, k, v)

    y = lax.dot_general(ctx, Wo.astype(jnp.bfloat16),
                        (((1,), (0,)), ((), ())),
                        preferred_element_type=jnp.float32)
    return y.reshape(B, Sq, D)


# device time: 227668 ns/iter; 1.5244x vs baseline; 1.5244x over previous
import jax
import jax.numpy as jnp
from jax import lax
from jax.experimental import pallas as pl
from jax.experimental.pallas import tpu as pltpu

N_DEV = 4
SCALE = 0.08838834764831843
BLK = 64


def kernel(x, Wq, K_ext, V_ext, Wo):
    B, Sq, D = x.shape
    _, Skv_l, Hq, Dh = K_ext.shape
    QT = 256
    n_qt = Sq // QT

    q = (lax.dot_general(x.reshape(Sq, D).astype(jnp.bfloat16),
                         Wq.astype(jnp.bfloat16),
                         (((1,), (0,)), ((), ())),
                         preferred_element_type=jnp.float32)
         * SCALE).astype(jnp.bfloat16)
    k = jnp.transpose(K_ext.reshape(Skv_l, Hq, Dh), (1, 0, 2)).astype(jnp.bfloat16)
    v = jnp.transpose(V_ext.reshape(Skv_l, Hq, Dh), (1, 0, 2)).astype(jnp.bfloat16)

    my_idx = lax.axis_index("i")
    qb = jnp.arange(Sq, dtype=jnp.int32)[:, None] // BLK
    kb = jnp.arange(Skv_l, dtype=jnp.int32)[None, :] // BLK + my_idx * (Skv_l // BLK)
    keep = (qb == kb) | (kb == 0) | ((qb + kb) % 3 == 0)
    bias = jnp.where(keep, 0.0, -1e9).astype(jnp.bfloat16)

    def body(q_ref, k_ref, v_ref, bias_ref, out_ref,
             o_comm, ml_comm, so, ro, sml, rml):
        my = lax.axis_index("i")
        left = lax.rem(my + (N_DEV - 1), N_DEV)
        right = lax.rem(my + 1, N_DEV)

        for h in range(Hq):
            def attn_step(t, carry, h=h):
                rows = pl.ds(t * QT, QT)
                q_t = q_ref[rows, h * Dh:(h + 1) * Dh]
                s = lax.dot_general(q_t, k_ref[h], (((1,), (1,)), ((), ())),
                                    preferred_element_type=jnp.float32)
                s = s + bias_ref[rows, :].astype(jnp.float32)
                m = jnp.max(s, axis=1, keepdims=True)
                p = jnp.exp(s - m)
                l = jnp.sum(p, axis=1, keepdims=True)
                o = lax.dot_general(p.astype(jnp.bfloat16), v_ref[h],
                                    (((1,), (0,)), ((), ())),
                                    preferred_element_type=jnp.float32)
                o_comm[0, rows, h * Dh:(h + 1) * Dh] = o.astype(jnp.bfloat16)
                ml_comm[0, rows, h:h + 1] = m
                ml_comm[0, rows, Hq + h:Hq + h + 1] = l
                return carry

            lax.fori_loop(0, n_qt, attn_step, 0)

        barrier_sem = pltpu.get_barrier_semaphore()
        for nbr in (left, right):
            pl.semaphore_signal(barrier_sem, inc=1, device_id=(nbr,),
                                device_id_type=pl.DeviceIdType.MESH)
        pl.semaphore_wait(barrier_sem, 2)

        r1 = [
            pltpu.make_async_remote_copy(
                src_ref=o_comm.at[0], dst_ref=o_comm.at[1],
                send_sem=so.at[0], recv_sem=ro.at[1],
                device_id=(right,), device_id_type=pl.DeviceIdType.MESH),
            pltpu.make_async_remote_copy(
                src_ref=o_comm.at[0], dst_ref=o_comm.at[2],
                send_sem=so.at[1], recv_sem=ro.at[2],
                device_id=(left,), device_id_type=pl.DeviceIdType.MESH),
            pltpu.make_async_remote_copy(
                src_ref=ml_comm.at[0], dst_ref=ml_comm.at[1],
                send_sem=sml.at[0], recv_sem=rml.at[1],
                device_id=(right,), device_id_type=pl.DeviceIdType.MESH),
            pltpu.make_async_remote_copy(
                src_ref=ml_comm.at[0], dst_ref=ml_comm.at[2],
                send_sem=sml.at[1], recv_sem=rml.at[2],
                device_id=(left,), device_id_type=pl.DeviceIdType.MESH),
        ]
        for r in r1:
            r.start()
        for r in r1:
            r.wait()

        HS = Sq // 2
        top = pl.ds(0, HS)
        bot = pl.ds(HS, HS)
        r2 = [
            pltpu.make_async_remote_copy(
                src_ref=o_comm.at[1, top], dst_ref=o_comm.at[3, top],
                send_sem=so.at[2], recv_sem=ro.at[3],
                device_id=(right,), device_id_type=pl.DeviceIdType.MESH),
            pltpu.make_async_remote_copy(
                src_ref=o_comm.at[2, bot], dst_ref=o_comm.at[3, bot],
                send_sem=so.at[3], recv_sem=ro.at[0],
                device_id=(left,), device_id_type=pl.DeviceIdType.MESH),
            pltpu.make_async_remote_copy(
                src_ref=ml_comm.at[1, top], dst_ref=ml_comm.at[3, top],
                send_sem=sml.at[2], recv_sem=rml.at[3],
                device_id=(right,), device_id_type=pl.DeviceIdType.MESH),
            pltpu.make_async_remote_copy(
                src_ref=ml_comm.at[2, bot], dst_ref=ml_comm.at[3, bot],
                send_sem=sml.at[3], recv_sem=rml.at[0],
                device_id=(left,), device_id_type=pl.DeviceIdType.MESH),
        ]
        for r in r2:
            r.start()
        for r in r2:
            r.wait()

        expand = (lax.broadcasted_iota(jnp.int32, (Hq, D), 1) // Dh ==
                  lax.broadcasted_iota(jnp.int32, (Hq, D), 0)
                  ).astype(jnp.float32)

        def comb_step(t, carry):
            rows = pl.ds(t * QT, QT)
            ms = [ml_comm[s_, rows, 0:Hq] for s_ in range(N_DEV)]
            M = jnp.maximum(jnp.maximum(ms[0], ms[1]),
                            jnp.maximum(ms[2], ms[3]))
            O = None
            L = None
            for s_ in range(N_DEV):
                w = jnp.exp(ms[s_] - M)
                lw = ml_comm[s_, rows, Hq:2 * Hq] * w
                wx = lax.dot_general(w, expand, (((1,), (0,)), ((), ())),
                                     preferred_element_type=jnp.float32)
                ow = o_comm[s_, rows, :].astype(jnp.float32) * wx
                L = lw if L is None else L + lw
                O = ow if O is None else O + ow
            Lx = lax.dot_general(L, expand, (((1,), (0,)), ((), ())),
                                 preferred_element_type=jnp.float32)
            out_ref[rows, :] = (O / Lx).astype(jnp.bfloat16)
            return carry

        lax.fori_loop(0, n_qt, comb_step, 0)

    ctx = pl.pallas_call(
        body,
        out_shape=jax.ShapeDtypeStruct((Sq, D), jnp.bfloat16),
        in_specs=[pl.BlockSpec(memory_space=pltpu.VMEM)] * 4,
        out_specs=pl.BlockSpec(memory_space=pltpu.VMEM),
        scratch_shapes=[
            pltpu.VMEM((N_DEV, Sq, D), jnp.bfloat16),
            pltpu.VMEM((N_DEV, Sq, 2 * Hq), jnp.float32),
            pltpu.SemaphoreType.DMA((N_DEV,)),
            pltpu.SemaphoreType.DMA((N_DEV,)),
            pltpu.SemaphoreType.DMA((N_DEV,)),
            pltpu.SemaphoreType.DMA((N_DEV,)),
        ],
        compiler_params=pltpu.CompilerParams(
            collective_id=0,
            vmem_limit_bytes=50 * 1024 * 1024,
        ),
    )(q, k, v, bias)

    y = lax.dot_general(ctx, Wo.astype(jnp.bfloat16),
                        (((1,), (0,)), ((), ())),
                        preferred_element_type=jnp.float32)
    return y.reshape(B, Sq, D)


# device time: 197678 ns/iter; 1.7557x vs baseline; 1.1517x over previous
import jax
import jax.numpy as jnp
from jax import lax
from jax.experimental import pallas as pl
from jax.experimental.pallas import tpu as pltpu

N_DEV = 4
SCALE = 0.08838834764831843
BLK = 64


def kernel(x, Wq, K_ext, V_ext, Wo):
    B, Sq, D = x.shape
    _, Skv_l, Hq, Dh = K_ext.shape
    QT = 256
    HS = Sq // 2
    n_qt_h = HS // QT

    q = (lax.dot_general(x.reshape(Sq, D).astype(jnp.bfloat16),
                         Wq.astype(jnp.bfloat16),
                         (((1,), (0,)), ((), ())),
                         preferred_element_type=jnp.float32)
         * SCALE).astype(jnp.bfloat16)
    k = jnp.transpose(K_ext.reshape(Skv_l, Hq, Dh), (1, 0, 2)).astype(jnp.bfloat16)
    v = jnp.transpose(V_ext.reshape(Skv_l, Hq, Dh), (1, 0, 2)).astype(jnp.bfloat16)

    my_idx = lax.axis_index("i")
    qb = jnp.arange(Sq, dtype=jnp.int32)[:, None] // BLK
    kb = jnp.arange(Skv_l, dtype=jnp.int32)[None, :] // BLK + my_idx * (Skv_l // BLK)
    keep = (qb == kb) | (kb == 0) | ((qb + kb) % 3 == 0)
    bias = jnp.where(keep, 0.0, -1e9).astype(jnp.bfloat16)

    def body(q_ref, k_ref, v_ref, bias_ref, out_ref,
             o_comm, ml_comm, so, ro, sml, rml):
        my = lax.axis_index("i")
        left = lax.rem(my + (N_DEV - 1), N_DEV)
        right = lax.rem(my + 1, N_DEV)
        RIGHT = ((right,), pl.DeviceIdType.MESH)
        LEFT = ((left,), pl.DeviceIdType.MESH)

        barrier_sem = pltpu.get_barrier_semaphore()
        for nbr in (left, right):
            pl.semaphore_signal(barrier_sem, inc=1, device_id=(nbr,),
                                device_id_type=pl.DeviceIdType.MESH)
        pl.semaphore_wait(barrier_sem, 2)

        def rdma(buf, src_slot, dst_slot, rows, ssem, rsem, dev):
            return pltpu.make_async_remote_copy(
                src_ref=buf.at[src_slot, rows], dst_ref=buf.at[dst_slot, rows],
                send_sem=ssem, recv_sem=rsem,
                device_id=dev[0], device_id_type=dev[1])

        def attn_half(half):
            base = half * HS
            for h in range(Hq):
                def attn_step(t, carry, h=h):
                    rows = pl.ds(base + t * QT, QT)
                    q_t = q_ref[rows, h * Dh:(h + 1) * Dh]
                    s = lax.dot_general(q_t, k_ref[h],
                                        (((1,), (1,)), ((), ())),
                                        preferred_element_type=jnp.float32)
                    s = s + bias_ref[rows, :].astype(jnp.float32)
                    m = jnp.max(s, axis=1, keepdims=True)
                    p = jnp.exp(s - m)
                    l = jnp.sum(p, axis=1, keepdims=True)
                    o = lax.dot_general(p.astype(jnp.bfloat16), v_ref[h],
                                        (((1,), (0,)), ((), ())),
                                        preferred_element_type=jnp.float32)
                    o_comm[0, rows, h * Dh:(h + 1) * Dh] = o.astype(jnp.bfloat16)
                    ml_comm[0, rows, h:h + 1] = m
                    ml_comm[0, rows, Hq + h:Hq + h + 1] = l
                    return carry

                lax.fori_loop(0, n_qt_h, attn_step, 0)

        top = pl.ds(0, HS)
        bot = pl.ds(HS, HS)

        attn_half(0)
        r1 = [
            rdma(o_comm, 0, 1, top, so.at[0], ro.at[0], RIGHT),
            rdma(o_comm, 0, 2, top, so.at[1], ro.at[2], LEFT),
            rdma(ml_comm, 0, 1, top, sml.at[0], rml.at[0], RIGHT),
            rdma(ml_comm, 0, 2, top, sml.at[1], rml.at[2], LEFT),
        ]
        for r in r1:
            r.start()

        attn_half(1)
        r1b = [
            rdma(o_comm, 0, 1, bot, so.at[2], ro.at[1], RIGHT),
            rdma(o_comm, 0, 2, bot, so.at[3], ro.at[3], LEFT),
            rdma(ml_comm, 0, 1, bot, sml.at[2], rml.at[1], RIGHT),
            rdma(ml_comm, 0, 2, bot, sml.at[3], rml.at[3], LEFT),
        ]
        for r in r1b:
            r.start()

        r1[0].wait_recv()
        r1[2].wait_recv()
        r2r = [
            rdma(o_comm, 1, 3, top, so.at[4], ro.at[4], RIGHT),
            rdma(ml_comm, 1, 3, top, sml.at[4], rml.at[4], RIGHT),
        ]
        for r in r2r:
            r.start()
        r1b[1].wait_recv()
        r1b[3].wait_recv()
        r2l = [
            rdma(o_comm, 2, 3, bot, so.at[5], ro.at[5], LEFT),
            rdma(ml_comm, 2, 3, bot, sml.at[5], rml.at[5], LEFT),
        ]
        for r in r2l:
            r.start()

        for r in (r1b[0], r1b[2], r1[1], r1[3]):
            r.wait_recv()
        for r in r1 + r1b:
            r.wait_send()

        expand = (lax.broadcasted_iota(jnp.int32, (Hq, D), 1) // Dh ==
                  lax.broadcasted_iota(jnp.int32, (Hq, D), 0)
                  ).astype(jnp.float32)

        def pre_step(t, carry):
            rows = pl.ds(t * QT, QT)
            ms = [ml_comm[s_, rows, 0:Hq] for s_ in range(3)]
            M = jnp.maximum(jnp.maximum(ms[0], ms[1]), ms[2])
            O = None
            L = None
            for s_ in range(3):
                w = jnp.exp(ms[s_] - M)
                lw = ml_comm[s_, rows, Hq:2 * Hq] * w
                wx = lax.dot_general(w, expand, (((1,), (0,)), ((), ())),
                                     preferred_element_type=jnp.float32)
                ow = o_comm[s_, rows, :].astype(jnp.float32) * wx
                L = lw if L is None else L + lw
                O = ow if O is None else O + ow
            o_comm[0, rows, :] = O.astype(jnp.bfloat16)
            ml_comm[0, rows, 0:Hq] = M
            ml_comm[0, rows, Hq:2 * Hq] = L
            return carry

        lax.fori_loop(0, Sq // QT, pre_step, 0)

        for r in r2r + r2l:
            r.wait_recv()

        def comb_step(t, carry):
            rows = pl.ds(t * QT, QT)
            m0 = ml_comm[0, rows, 0:Hq]
            m3 = ml_comm[3, rows, 0:Hq]
            M = jnp.maximum(m0, m3)
            w0 = jnp.exp(m0 - M)
            w3 = jnp.exp(m3 - M)
            L = ml_comm[0, rows, Hq:2 * Hq] * w0 \
                + ml_comm[3, rows, Hq:2 * Hq] * w3
            w0x = lax.dot_general(w0, expand, (((1,), (0,)), ((), ())),
                                  preferred_element_type=jnp.float32)
            w3x = lax.dot_general(w3, expand, (((1,), (0,)), ((), ())),
                                  preferred_element_type=jnp.float32)
            O = o_comm[0, rows, :].astype(jnp.float32) * w0x \
                + o_comm[3, rows, :].astype(jnp.float32) * w3x
            Lx = lax.dot_general(L, expand, (((1,), (0,)), ((), ())),
                                 preferred_element_type=jnp.float32)
            out_ref[rows, :] = (O / Lx).astype(jnp.bfloat16)
            return carry

        lax.fori_loop(0, Sq // QT, comb_step, 0)

        for r in r2r + r2l:
            r.wait_send()

    ctx = pl.pallas_call(
        body,
        out_shape=jax.ShapeDtypeStruct((Sq, D), jnp.bfloat16),
        in_specs=[pl.BlockSpec(memory_space=pltpu.VMEM)] * 4,
        out_specs=pl.BlockSpec(memory_space=pltpu.VMEM),
        scratch_shapes=[
            pltpu.VMEM((N_DEV, Sq, D), jnp.bfloat16),
            pltpu.VMEM((N_DEV, Sq, 2 * Hq), jnp.float32),
            pltpu.SemaphoreType.DMA((8,)),
            pltpu.SemaphoreType.DMA((8,)),
            pltpu.SemaphoreType.DMA((8,)),
            pltpu.SemaphoreType.DMA((8,)),
        ],
        compiler_params=pltpu.CompilerParams(
            collective_id=0,
            vmem_limit_bytes=50 * 1024 * 1024,
        ),
    )(q, k, v, bias)

    y = lax.dot_general(ctx, Wo.astype(jnp.bfloat16),
                        (((1,), (0,)), ((), ())),
                        preferred_element_type=jnp.float32)
    return y.reshape(B, Sq, D)
